# baseline (speedup 1.0000x reference)
import jax
import jax.numpy as jnp
from jax.experimental import pallas as pl

_TILE = 1000


def _mm_body(x_ref, m_ref, o_ref):
    o_ref[...] = jnp.dot(x_ref[...], m_ref[...],
                         preferred_element_type=jnp.float32)


def _pmm(x, m):
    n, k = x.shape
    _, c = m.shape
    return pl.pallas_call(
        _mm_body,
        grid=(n // _TILE,),
        in_specs=[
            pl.BlockSpec((_TILE, k), lambda i: (i, 0)),
            pl.BlockSpec((k, c), lambda i: (0, 0)),
        ],
        out_specs=pl.BlockSpec((_TILE, c), lambda i: (i, 0)),
        out_shape=jax.ShapeDtypeStruct((n, c), jnp.float32),
    )(x, m)


def _pad_cols(m, c):
    return jnp.pad(m, ((0, 0), (0, c - m.shape[1])))


def _seg_softmax_agg(a_l, a_r, xl, row, col, n_tgt):
    e = jax.nn.leaky_relu(a_l[col] + a_r[row], 0.2)
    emax = jax.ops.segment_max(e, row, num_segments=n_tgt)
    emax = jnp.where(jnp.isfinite(emax), emax, 0.0)
    ex = jnp.exp(e - emax[row])
    denom = jax.ops.segment_sum(ex, row, num_segments=n_tgt)
    alpha = ex / (denom[row] + 1e-16)
    return jax.ops.segment_sum(alpha[:, None] * xl[col], row,
                               num_segments=n_tgt)


def kernel(conc_x, item_x, stu_x, cc_edge_index, ic_item_idx, ic_conc_idx,
           si_stu_idx, si_item_idx,
           W_cc, al_cc, ar_cc, W_ic, al_ic, ar_ic,
           al_ecc, ar_ecc, al_eic, ar_eic,
           W_ci, al_ci, ar_ci, W_si, al_si, ar_si,
           Wa_ci, Wa_si, W_is, al_is, ar_is):
    CONC, ITEM, STU = conc_x.shape[0], item_x.shape[0], stu_x.shape[0]
    D = conc_x.shape[1]
    cc_row, cc_col = cc_edge_index[0], cc_edge_index[1]

    wl_cc = W_cc.T @ al_cc
    wr_cc = W_cc.T @ ar_cc
    wl_ic = W_ic.T @ al_ic
    wr_ic = W_ic.T @ ar_ic
    wl_ci = W_ci.T @ al_ci
    wr_ci = W_ci.T @ ar_ci
    wl_si = W_si.T @ al_si
    wr_si = W_si.T @ ar_si
    wl_is = W_is.T @ al_is
    wr_is = W_is.T @ ar_is

    m_conc = jnp.concatenate([
        W_cc.T, W_ci.T,
        wl_cc[:, None], wr_cc[:, None], wr_ic[:, None], wl_ci[:, None],
        ar_ecc[:, None], ar_eic[:, None],
    ], axis=1)
    m_conc = _pad_cols(m_conc, 384)
    yc = _pmm(conc_x, m_conc)
    xl_cc = yc[:, 0:D]
    xl_ci = yc[:, D:2 * D]
    alv_cc = yc[:, 2 * D]
    arv_cc = yc[:, 2 * D + 1]
    arv_ic = yc[:, 2 * D + 2]
    alv_ci = yc[:, 2 * D + 3]
    arv_ecc = yc[:, 2 * D + 4]
    arv_eic = yc[:, 2 * D + 5]

    m_item = _pad_cols(jnp.concatenate(
        [W_ic.T, wl_ic[:, None], wr_ci[:, None], wr_si[:, None]], axis=1),
        256)
    yi = _pmm(item_x, m_item)
    xl_ic = yi[:, 0:D]
    alv_ic = yi[:, D]
    arv_ci = yi[:, D + 1]
    arv_si = yi[:, D + 2]

    m_stu = _pad_cols(jnp.concatenate(
        [W_si.T, wl_si[:, None], wr_is[:, None]], axis=1), 256)
    ys = _pmm(stu_x, m_stu)
    xl_si = ys[:, 0:D]
    alv_si = ys[:, D]
    arv_is = ys[:, D + 1]

    c_cc = _seg_softmax_agg(alv_cc, arv_cc, xl_cc, cc_row, cc_col, CONC)
    c_ic = _seg_softmax_agg(alv_ic, arv_ic, xl_ic, ic_conc_idx, ic_item_idx,
                            CONC)
    m_e = _pad_cols(jnp.concatenate(
        [jnp.concatenate([al_ecc[:, None], jnp.zeros((D, 1))], axis=1),
         jnp.concatenate([jnp.zeros((D, 1)), al_eic[:, None]], axis=1)],
        axis=0), 128)
    cat_e = jnp.concatenate([c_cc, c_ic], axis=1)
    ye = _pmm(cat_e, m_e)
    al2 = ye[:, 0:2]
    ar2 = jnp.stack([arv_ecc, arv_eic], axis=1)
    e2 = jax.nn.leaky_relu(al2[cc_col] + ar2[cc_row], 0.2)
    emax2 = jax.ops.segment_max(e2, cc_row, num_segments=CONC)
    emax2 = jnp.where(jnp.isfinite(emax2), emax2, 0.0)
    ex2 = jnp.exp(e2 - emax2[cc_row])
    den2 = jax.ops.segment_sum(ex2, cc_row, num_segments=CONC)
    alpha2 = ex2 / (den2[cc_row] + 1e-16)
    msg2 = cat_e[cc_col].reshape(-1, 2, D) * alpha2[:, :, None]
    out2 = jax.ops.segment_sum(msg2, cc_row, num_segments=CONC)
    conc_fused = conc_x + out2[:, 0, :] + out2[:, 1, :]

    i_ci = _seg_softmax_agg(alv_ci, arv_ci, xl_ci, ic_item_idx, ic_conc_idx,
                            ITEM)
    i_si = _seg_softmax_agg(alv_si, arv_si, xl_si, si_item_idx, si_stu_idx,
                            ITEM)
    item_fused = item_x + i_ci + i_si

    m_if = _pad_cols(jnp.concatenate([W_is.T, wl_is[:, None]], axis=1), 256)
    yf = _pmm(item_fused, m_if)
    xl_is = yf[:, 0:D]
    alv_is = yf[:, D]
    s_is = _seg_softmax_agg(alv_is, arv_is, xl_is, si_stu_idx, si_item_idx,
                            STU)
    stu_fused = stu_x + s_is
    return (conc_fused, item_fused, stu_fused)


# speedup vs baseline: 1.4084x; 1.4084x over previous
import jax
import jax.numpy as jnp
from jax.experimental import pallas as pl

_TILE = 1000


def _mm_body(x_ref, m_ref, o_ref):
    o_ref[...] = jnp.dot(x_ref[...], m_ref[...],
                         preferred_element_type=jnp.float32)


def _pmm(x, m):
    n, k = x.shape
    _, c = m.shape
    return pl.pallas_call(
        _mm_body,
        grid=(n // _TILE,),
        in_specs=[
            pl.BlockSpec((_TILE, k), lambda i: (i, 0)),
            pl.BlockSpec((k, c), lambda i: (0, 0)),
        ],
        out_specs=pl.BlockSpec((_TILE, c), lambda i: (i, 0)),
        out_shape=jax.ShapeDtypeStruct((n, c), jnp.float32),
    )(x, m)


def _pad_cols(m, c):
    return jnp.pad(m, ((0, 0), (0, c - m.shape[1])))


def _seg_softmax_agg(a_l, a_r, xl, row, col, n_tgt):
    e = jax.nn.leaky_relu(a_l[col] + a_r[row], 0.2)
    emax = jax.ops.segment_max(e, row, num_segments=n_tgt)
    emax = jnp.where(jnp.isfinite(emax), emax, 0.0)
    ex = jnp.exp(e - emax[row])
    denom = jax.ops.segment_sum(ex, row, num_segments=n_tgt)
    alpha = ex / (denom[row] + 1e-16)
    return jax.ops.segment_sum(alpha[:, None] * xl[col], row,
                               num_segments=n_tgt)


def kernel(conc_x, item_x, stu_x, cc_edge_index, ic_item_idx, ic_conc_idx,
           si_stu_idx, si_item_idx,
           W_cc, al_cc, ar_cc, W_ic, al_ic, ar_ic,
           al_ecc, ar_ecc, al_eic, ar_eic,
           W_ci, al_ci, ar_ci, W_si, al_si, ar_si,
           Wa_ci, Wa_si, W_is, al_is, ar_is):
    CONC, ITEM, STU = conc_x.shape[0], item_x.shape[0], stu_x.shape[0]
    D = conc_x.shape[1]
    cc_row, cc_col = cc_edge_index[0], cc_edge_index[1]

    wl_cc = W_cc.T @ al_cc
    wr_cc = W_cc.T @ ar_cc
    wl_ic = W_ic.T @ al_ic
    wr_ic = W_ic.T @ ar_ic
    wl_ci = W_ci.T @ al_ci
    wr_ci = W_ci.T @ ar_ci
    wl_si = W_si.T @ al_si
    wr_si = W_si.T @ ar_si
    wl_is = W_is.T @ al_is
    wr_is = W_is.T @ ar_is

    m_conc = jnp.concatenate([
        W_cc.T, W_ci.T,
        wl_cc[:, None], wr_cc[:, None], wr_ic[:, None], wl_ci[:, None],
        ar_ecc[:, None], ar_eic[:, None],
    ], axis=1)
    m_conc = _pad_cols(m_conc, 384)
    yc = _pmm(conc_x, m_conc)
    xl_cc = yc[:, 0:D]
    xl_ci = yc[:, D:2 * D]
    alv_cc = yc[:, 2 * D]
    arv_cc = yc[:, 2 * D + 1]
    arv_ic = yc[:, 2 * D + 2]
    alv_ci = yc[:, 2 * D + 3]
    arv_ecc = yc[:, 2 * D + 4]
    arv_eic = yc[:, 2 * D + 5]

    m_item = _pad_cols(jnp.concatenate(
        [W_ic.T, wl_ic[:, None], wr_ci[:, None], wr_si[:, None]], axis=1),
        256)
    yi = _pmm(item_x, m_item)
    xl_ic = yi[:, 0:D]
    alv_ic = yi[:, D]
    arv_ci = yi[:, D + 1]
    arv_si = yi[:, D + 2]

    m_stu = _pad_cols(jnp.concatenate(
        [W_si.T, wl_si[:, None], wr_is[:, None]], axis=1), 256)
    ys = _pmm(stu_x, m_stu)
    xl_si = ys[:, 0:D]
    alv_si = ys[:, D]
    arv_is = ys[:, D + 1]

    c_cc = _seg_softmax_agg(alv_cc, arv_cc, xl_cc, cc_row, cc_col, CONC)
    c_ic = _seg_softmax_agg(alv_ic, arv_ic, xl_ic, ic_conc_idx, ic_item_idx,
                            CONC)
    m_e = _pad_cols(jnp.concatenate(
        [jnp.concatenate([al_ecc[:, None], jnp.zeros((D, 1))], axis=1),
         jnp.concatenate([jnp.zeros((D, 1)), al_eic[:, None]], axis=1)],
        axis=0), 128)
    cat_e = jnp.concatenate([c_cc, c_ic], axis=1)
    ye = _pmm(cat_e, m_e)
    al2 = ye[:, 0:2]
    ar2 = jnp.stack([arv_ecc, arv_eic], axis=1)
    e2 = jax.nn.leaky_relu(al2[cc_col] + ar2[cc_row], 0.2)
    emax2 = jax.ops.segment_max(e2, cc_row, num_segments=CONC)
    emax2 = jnp.where(jnp.isfinite(emax2), emax2, 0.0)
    ex2 = jnp.exp(e2 - emax2[cc_row])
    den2 = jax.ops.segment_sum(ex2, cc_row, num_segments=CONC)
    alpha2 = ex2 / (den2[cc_row] + 1e-16)
    aw2 = jnp.concatenate([jnp.broadcast_to(alpha2[:, 0:1], (alpha2.shape[0], D)),
                           jnp.broadcast_to(alpha2[:, 1:2], (alpha2.shape[0], D))],
                          axis=1)
    out2 = jax.ops.segment_sum(cat_e[cc_col] * aw2, cc_row, num_segments=CONC)
    conc_fused = conc_x + out2[:, 0:D] + out2[:, D:2 * D]

    i_ci = _seg_softmax_agg(alv_ci, arv_ci, xl_ci, ic_item_idx, ic_conc_idx,
                            ITEM)
    i_si = _seg_softmax_agg(alv_si, arv_si, xl_si, si_item_idx, si_stu_idx,
                            ITEM)
    item_fused = item_x + i_ci + i_si

    m_if = _pad_cols(jnp.concatenate([W_is.T, wl_is[:, None]], axis=1), 256)
    yf = _pmm(item_fused, m_if)
    xl_is = yf[:, 0:D]
    alv_is = yf[:, D]
    s_is = _seg_softmax_agg(alv_is, arv_is, xl_is, si_stu_idx, si_item_idx,
                            STU)
    stu_fused = stu_x + s_is
    return (conc_fused, item_fused, stu_fused)
